# baseline (speedup 1.0000x reference)
import functools

import jax
import jax.numpy as jnp
from jax import lax
from jax.experimental import pallas as pl
from jax.experimental.pallas import tpu as pltpu
from jax.experimental.pallas import tpu_sc as plsc

NUM_CORES = 2
NUM_SUBCORES = 16
NW = NUM_CORES * NUM_SUBCORES
LANES = 16
BATCH = 16384
FACTORS = 64
B_PER_W = BATCH // NW
CHUNK = 128
NCHUNK = B_PER_W // CHUNK


def _body(users_hbm, items_hbm, gu_hbm, gi_hbm, bi_hbm,
          xui_out, beta_out, gu_out, gi_out,
          uidx_v, iidx_v, gu_v, gi_v, bv, xui_v, sem):
  wid = lax.axis_index("s") * NUM_CORES + lax.axis_index("c")
  base = wid * B_PER_W

  pltpu.sync_copy(users_hbm.at[pl.ds(wid * NCHUNK, NCHUNK)], uidx_v)
  pltpu.sync_copy(items_hbm.at[pl.ds(wid * NCHUNK, NCHUNK)], iidx_v)

  copies = []
  for j in range(NCHUNK):
    copies.append(pltpu.async_copy(
        gu_hbm.at[uidx_v.at[j]], gu_v.at[pl.ds(j * CHUNK, CHUNK)], sem))
    copies.append(pltpu.async_copy(
        gi_hbm.at[iidx_v.at[j]], gi_v.at[pl.ds(j * CHUNK, CHUNK)], sem))
    copies.append(pltpu.async_copy(
        bi_hbm.at[iidx_v.at[j]], bv.at[pl.ds(j * CHUNK, CHUNK)], sem))
  for c in copies:
    c.wait()

  lane = lax.iota(jnp.int32, LANES)

  def group(g, _):
    res = jnp.zeros((LANES,), jnp.float32)
    for j in range(LANES):
      r = g * LANES + j
      acc = gu_v[r, pl.ds(0, LANES)] * gi_v[r, pl.ds(0, LANES)]
      for c in range(1, FACTORS // LANES):
        acc += (gu_v[r, pl.ds(c * LANES, LANES)] *
                gi_v[r, pl.ds(c * LANES, LANES)])
      res = jnp.where(lane == j, jnp.sum(acc), res)
    xui_v[pl.ds(g * LANES, LANES)] = res + bv[pl.ds(g * LANES, LANES)]
    return 0

  lax.fori_loop(0, B_PER_W // LANES, group, 0)

  pltpu.sync_copy(gu_v, gu_out.at[pl.ds(base, B_PER_W)])
  pltpu.sync_copy(gi_v, gi_out.at[pl.ds(base, B_PER_W)])
  pltpu.sync_copy(bv, beta_out.at[pl.ds(base, B_PER_W)])
  pltpu.sync_copy(xui_v, xui_out.at[pl.ds(base, B_PER_W)])


@jax.jit
def _run(users2, items2, Gu, Gi, bi_flat):
  mesh = plsc.VectorSubcoreMesh(core_axis_name="c", subcore_axis_name="s")
  f = pl.kernel(
      _body,
      out_type=(
          jax.ShapeDtypeStruct((BATCH,), jnp.float32),
          jax.ShapeDtypeStruct((BATCH,), jnp.float32),
          jax.ShapeDtypeStruct((BATCH, FACTORS), jnp.float32),
          jax.ShapeDtypeStruct((BATCH, FACTORS), jnp.float32),
      ),
      mesh=mesh,
      compiler_params=pltpu.CompilerParams(
          needs_layout_passes=False, use_tc_tiling_on_sc=False),
      scratch_types=[
          pltpu.VMEM((NCHUNK, CHUNK), jnp.int32),
          pltpu.VMEM((NCHUNK, CHUNK), jnp.int32),
          pltpu.VMEM((B_PER_W, FACTORS), jnp.float32),
          pltpu.VMEM((B_PER_W, FACTORS), jnp.float32),
          pltpu.VMEM((B_PER_W,), jnp.float32),
          pltpu.VMEM((B_PER_W,), jnp.float32),
          pltpu.SemaphoreType.DMA,
      ],
  )
  return f(users2, items2, Gu, Gi, bi_flat)


def kernel(users_indices, items_indices, Gu, Gi, Bi):
  users2 = users_indices.astype(jnp.int32).reshape(BATCH // CHUNK, CHUNK)
  items2 = items_indices.astype(jnp.int32).reshape(BATCH // CHUNK, CHUNK)
  bi_flat = Bi.reshape(Bi.shape[0])
  xui, beta_i, gamma_u, gamma_i = _run(users2, items2, Gu, Gi, bi_flat)
  return (xui, beta_i, gamma_u, gamma_i)


# speedup vs baseline: 1.4877x; 1.4877x over previous
import functools

import jax
import jax.numpy as jnp
from jax import lax
from jax.experimental import pallas as pl
from jax.experimental.pallas import tpu as pltpu
from jax.experimental.pallas import tpu_sc as plsc

NUM_CORES = 2
NUM_SUBCORES = 16
NW = NUM_CORES * NUM_SUBCORES
LANES = 16
BATCH = 16384
FACTORS = 64
B_PER_W = BATCH // NW
CHUNK = 128
NCHUNK = B_PER_W // CHUNK
PASS_ROWS = 256
NPASS = B_PER_W // PASS_ROWS


def _body(users_hbm, items_hbm, gu_hbm, gi_hbm, bi_hbm,
          xui_out, beta_out, gu_out, gi_out,
          uidx_v, iidx_v, gu_v, gi_v, bv, xui_v, sem, semb):
  wid = lax.axis_index("s") * NUM_CORES + lax.axis_index("c")
  base = wid * B_PER_W

  pltpu.sync_copy(users_hbm.at[pl.ds(wid * NCHUNK, NCHUNK)], uidx_v)
  pltpu.sync_copy(items_hbm.at[pl.ds(wid * NCHUNK, NCHUNK)], iidx_v)

  bcopies = [
      pltpu.async_copy(bi_hbm.at[iidx_v.at[j]],
                       bv.at[pl.ds(j * CHUNK, CHUNK)], semb)
      for j in range(NCHUNK)
  ]

  lane = lax.iota(jnp.int32, LANES)

  for p in range(NPASS):
    row0 = p * PASS_ROWS

    for j in range(row0 // CHUNK, (row0 + PASS_ROWS) // CHUNK):
      def fire(g, _, j=j):
        uvec = uidx_v[j, pl.ds(g * LANES, LANES)]
        ivec = iidx_v[j, pl.ds(g * LANES, LANES)]
        r0 = (j * CHUNK - row0) + g * LANES
        for t in range(LANES):
          u = lax.squeeze(lax.slice(uvec, (t,), (t + 1,)), (0,))
          i = lax.squeeze(lax.slice(ivec, (t,), (t + 1,)), (0,))
          pltpu.async_copy(gu_hbm.at[pl.ds(u, 1)],
                           gu_v.at[pl.ds(r0 + t, 1)], sem)
          pltpu.async_copy(gi_hbm.at[pl.ds(i, 1)],
                           gi_v.at[pl.ds(r0 + t, 1)], sem)
        return 0

      lax.fori_loop(0, CHUNK // LANES, fire, 0)

    pltpu.make_async_copy(gu_hbm.at[pl.ds(0, PASS_ROWS)], gu_v, sem).wait()
    pltpu.make_async_copy(gi_hbm.at[pl.ds(0, PASS_ROWS)], gi_v, sem).wait()

    if p == 0:
      for c in bcopies:
        c.wait()

    def group(g, _):
      res = jnp.zeros((LANES,), jnp.float32)
      for j in range(LANES):
        r = g * LANES + j
        acc = gu_v[r, pl.ds(0, LANES)] * gi_v[r, pl.ds(0, LANES)]
        for c in range(1, FACTORS // LANES):
          acc += (gu_v[r, pl.ds(c * LANES, LANES)] *
                  gi_v[r, pl.ds(c * LANES, LANES)])
        res = jnp.where(lane == j, jnp.sum(acc), res)
      xui_v[pl.ds(row0 + g * LANES, LANES)] = (
          res + bv[pl.ds(row0 + g * LANES, LANES)])
      return 0

    lax.fori_loop(0, PASS_ROWS // LANES, group, 0)

    pltpu.sync_copy(gu_v, gu_out.at[pl.ds(base + row0, PASS_ROWS)])
    pltpu.sync_copy(gi_v, gi_out.at[pl.ds(base + row0, PASS_ROWS)])

  pltpu.sync_copy(bv, beta_out.at[pl.ds(base, B_PER_W)])
  pltpu.sync_copy(xui_v, xui_out.at[pl.ds(base, B_PER_W)])


@jax.jit
def _run(users2, items2, Gu, Gi, bi_flat):
  mesh = plsc.VectorSubcoreMesh(core_axis_name="c", subcore_axis_name="s")
  f = pl.kernel(
      _body,
      out_type=(
          jax.ShapeDtypeStruct((BATCH,), jnp.float32),
          jax.ShapeDtypeStruct((BATCH,), jnp.float32),
          jax.ShapeDtypeStruct((BATCH, FACTORS), jnp.float32),
          jax.ShapeDtypeStruct((BATCH, FACTORS), jnp.float32),
      ),
      mesh=mesh,
      compiler_params=pltpu.CompilerParams(needs_layout_passes=False),
      scratch_types=[
          pltpu.VMEM((NCHUNK, CHUNK), jnp.int32),
          pltpu.VMEM((NCHUNK, CHUNK), jnp.int32),
          pltpu.VMEM((PASS_ROWS, FACTORS), jnp.float32),
          pltpu.VMEM((PASS_ROWS, FACTORS), jnp.float32),
          pltpu.VMEM((B_PER_W,), jnp.float32),
          pltpu.VMEM((B_PER_W,), jnp.float32),
          pltpu.SemaphoreType.DMA,
          pltpu.SemaphoreType.DMA,
      ],
  )
  return f(users2, items2, Gu, Gi, bi_flat)


def kernel(users_indices, items_indices, Gu, Gi, Bi):
  users2 = users_indices.astype(jnp.int32).reshape(BATCH // CHUNK, CHUNK)
  items2 = items_indices.astype(jnp.int32).reshape(BATCH // CHUNK, CHUNK)
  bi_flat = Bi.reshape(Bi.shape[0])
  xui, beta_i, gamma_u, gamma_i = _run(users2, items2, Gu, Gi, bi_flat)
  return (xui, beta_i, gamma_u, gamma_i)
